# baseline (speedup 1.0000x reference)
import jax
import jax.numpy as jnp
from jax.experimental import pallas as pl
from jax.experimental.pallas import tpu as pltpu

N_Q = 8
HID = 1024
T = 2048
C = 512
NB = 8
ROWS = 32 * T
CH = ROWS // C
PER_SLAB = T // C


def _add_kernel(x_hbm, emb_hbm, o_hbm, emb_v, *bufs_and_sems):
    ibufs = bufs_and_sems[0:NB]
    obufs = bufs_and_sems[NB:2 * NB]
    sem_e = bufs_and_sems[2 * NB]
    sins = bufs_and_sems[2 * NB + 1:2 * NB + 1 + NB]
    souts = bufs_and_sems[2 * NB + 1 + NB:]

    def in_copy(b, i):
        return pltpu.make_async_copy(
            x_hbm.at[pl.ds(i * C, C), :], ibufs[b], sins[b])

    def out_copy(b, i):
        return pltpu.make_async_copy(
            obufs[b], o_hbm.at[pl.ds(i * C, C), :], souts[b])

    pltpu.make_async_copy(emb_hbm, emb_v, sem_e).start()
    for b in range(NB):
        in_copy(b, b).start()
    pltpu.make_async_copy(emb_hbm, emb_v, sem_e).wait()

    def outer(k, _):
        for b in range(NB):
            i = k * NB + b
            in_copy(b, i).wait()

            @pl.when(i >= NB)
            def _():
                out_copy(b, i - NB).wait()

            q = (i // PER_SLAB) % N_Q
            obufs[b][...] = ibufs[b][...] + emb_v[pl.ds(q, 1), :]
            out_copy(b, i).start()

            @pl.when(i + NB < CH)
            def _():
                in_copy(b, i + NB).start()

        return 0

    jax.lax.fori_loop(0, CH // NB, outer, 0)

    for b in range(NB):
        out_copy(b, CH - NB + b).wait()


def kernel(x, emb_table):
    b, q, t, h = x.shape
    xf = x.reshape(b * q * t, h)
    out = pl.pallas_call(
        _add_kernel,
        in_specs=[
            pl.BlockSpec(memory_space=pl.ANY),
            pl.BlockSpec(memory_space=pl.ANY),
        ],
        out_specs=pl.BlockSpec(memory_space=pl.ANY),
        out_shape=jax.ShapeDtypeStruct((b * q * t, h), x.dtype),
        scratch_shapes=(
            [pltpu.VMEM((N_Q, h), jnp.float32)]
            + [pltpu.VMEM((C, h), jnp.float32) for _ in range(2 * NB)]
            + [pltpu.SemaphoreType.DMA for _ in range(2 * NB + 1)]
        ),
    )(xf, emb_table)
    return out.reshape(b, q, t, h)


# speedup vs baseline: 1.0010x; 1.0010x over previous
import jax
import jax.numpy as jnp
from jax.experimental import pallas as pl
from jax.experimental.pallas import tpu as pltpu

N_Q = 8
HID = 1024
T = 2048
C = 2048
NB = 2
ROWS = 32 * T
CH = ROWS // C
PER_SLAB = T // C


def _add_kernel(x_hbm, emb_hbm, o_hbm, emb_v, *bufs_and_sems):
    ibufs = bufs_and_sems[0:NB]
    obufs = bufs_and_sems[NB:2 * NB]
    sem_e = bufs_and_sems[2 * NB]
    sins = bufs_and_sems[2 * NB + 1:2 * NB + 1 + NB]
    souts = bufs_and_sems[2 * NB + 1 + NB:]

    def in_copy(b, i):
        return pltpu.make_async_copy(
            x_hbm.at[pl.ds(i * C, C), :], ibufs[b], sins[b])

    def out_copy(b, i):
        return pltpu.make_async_copy(
            obufs[b], o_hbm.at[pl.ds(i * C, C), :], souts[b])

    pltpu.make_async_copy(emb_hbm, emb_v, sem_e).start()
    for b in range(NB):
        in_copy(b, b).start()
    pltpu.make_async_copy(emb_hbm, emb_v, sem_e).wait()

    def outer(k, _):
        for b in range(NB):
            i = k * NB + b
            in_copy(b, i).wait()

            @pl.when(i >= NB)
            def _():
                out_copy(b, i - NB).wait()

            q = (i // PER_SLAB) % N_Q
            obufs[b][...] = ibufs[b][...] + emb_v[pl.ds(q, 1), :]
            out_copy(b, i).start()

            @pl.when(i + NB < CH)
            def _():
                in_copy(b, i + NB).start()

        return 0

    jax.lax.fori_loop(0, CH // NB, outer, 0)

    for b in range(NB):
        out_copy(b, CH - NB + b).wait()


def kernel(x, emb_table):
    b, q, t, h = x.shape
    xf = x.reshape(b * q * t, h)
    out = pl.pallas_call(
        _add_kernel,
        in_specs=[
            pl.BlockSpec(memory_space=pl.ANY),
            pl.BlockSpec(memory_space=pl.ANY),
        ],
        out_specs=pl.BlockSpec(memory_space=pl.ANY),
        out_shape=jax.ShapeDtypeStruct((b * q * t, h), x.dtype),
        scratch_shapes=(
            [pltpu.VMEM((N_Q, h), jnp.float32)]
            + [pltpu.VMEM((C, h), jnp.float32) for _ in range(2 * NB)]
            + [pltpu.SemaphoreType.DMA for _ in range(2 * NB + 1)]
        ),
    )(xf, emb_table)
    return out.reshape(b, q, t, h)
